# baseline (speedup 1.0000x reference)
import functools

import jax
import jax.numpy as jnp
from jax import lax
from jax.experimental import pallas as pl
from jax.experimental.pallas import tpu as pltpu
from jax.experimental.pallas import tpu_sc as plsc

NC = 2
NS = 16
NW = NC * NS
L = 16

N = 10000
E = 160000
IN_CH = 256
HID_CH = 512

NPAD = 10240
SLAB = NPAD // NW
HALF = NPAD // NC
PADNODE = 10200
EPAD = 163840
ET2 = EPAD // NS
DROWS = 1280
DR_TILE = DROWS // NW


def _sc1_body(dst2d_hbm, deg_hbm, deg_sh, idx2d, ones_v, zer_v, deg_v):
    c = lax.axis_index("c")
    s = lax.axis_index("s")

    for i in range(128 // L):
        ones_v[pl.ds(i * L, L)] = jnp.ones((L,), jnp.float32)
    for i in range(640 // L):
        zer_v[pl.ds(i * L, L)] = jnp.zeros((L,), jnp.float32)

    pltpu.sync_copy(zer_v, deg_sh.at[pl.ds(s * 640, 640)])

    pltpu.sync_copy(
        dst2d_hbm.at[pl.ds(c * (DROWS // 2) + s * DR_TILE, DR_TILE)], idx2d)
    plsc.subcore_barrier()

    def deg_chunk(j, carry):
        pltpu.sync_copy(ones_v, deg_sh.at[idx2d.at[j]], add=True)
        return carry
    lax.fori_loop(0, DR_TILE, deg_chunk, 0)
    plsc.subcore_barrier()

    pltpu.sync_copy(deg_sh.at[pl.ds(s * 640, 640)], deg_v)
    pltpu.sync_copy(deg_v, deg_hbm.at[c, pl.ds(s * 640, 640)])


def _sc2_body(src_hbm, dst_hbm, xs_hbm, agg_hbm,
              srcb, dstb, gidx, didx, rows, sem):
    c = lax.axis_index("c")
    s = lax.axis_index("s")
    lo = c * HALF

    def zrow(r, carry):
        for t in range(IN_CH // L):
            rows[r, pl.ds(t * L, L)] = jnp.zeros((L,), jnp.float32)
        return carry
    lax.fori_loop(0, 128, zrow, 0)
    base = lo + s * SLAB
    for k, sz in ((0, 128), (128, 128), (256, 64)):
        pltpu.sync_copy(rows.at[pl.ds(0, sz)], agg_hbm.at[pl.ds(base + k, sz)])

    pltpu.sync_copy(src_hbm.at[pl.ds(s * ET2, ET2)], srcb)
    pltpu.sync_copy(dst_hbm.at[pl.ds(s * ET2, ET2)], dstb)
    plsc.subcore_barrier()

    def batch_body(kb, carry):
        k = kb * 128
        for t in range(128 // L):
            dv = dstb[pl.ds(k + t * L, L)]
            sv = srcb[pl.ds(k + t * L, L)]
            m = (dv >= lo) & (dv < lo + HALF)
            gidx[0, pl.ds(t * L, L)] = jnp.where(m, sv, PADNODE)
            didx[0, pl.ds(t * L, L)] = jnp.where(m, dv, lo)
        pltpu.async_copy(xs_hbm.at[gidx.at[0]], rows, sem).wait()
        pltpu.sync_copy(rows, agg_hbm.at[didx.at[0]], add=True)
        return carry
    lax.fori_loop(0, ET2 // 128, batch_body, 0)


_sc_mesh = plsc.VectorSubcoreMesh(core_axis_name="c", subcore_axis_name="s",
                                  num_cores=NC, num_subcores=NS)

_sc1 = functools.partial(
    pl.kernel,
    out_type=jax.ShapeDtypeStruct((NC, NPAD), jnp.float32),
    mesh=_sc_mesh,
    scratch_types=[
        pltpu.VMEM_SHARED((NPAD,), jnp.float32),
        pltpu.VMEM((DR_TILE, 128), jnp.int32),
        pltpu.VMEM((128,), jnp.float32),
        pltpu.VMEM((640,), jnp.float32),
        pltpu.VMEM((640,), jnp.float32),
    ],
)(_sc1_body)

_sc2 = functools.partial(
    pl.kernel,
    out_type=jax.ShapeDtypeStruct((NPAD, IN_CH), jnp.float32),
    mesh=_sc_mesh,
    scratch_types=[
        pltpu.VMEM((ET2,), jnp.int32),
        pltpu.VMEM((ET2,), jnp.int32),
        pltpu.VMEM((1, 128), jnp.int32),
        pltpu.VMEM((1, 128), jnp.int32),
        pltpu.VMEM((128, IN_CH), jnp.float32),
        pltpu.SemaphoreType.DMA,
    ],
)(_sc2_body)


def _tc_pre_body(deg_ref, x_ref, dis_ref, xs_ref):
    d = jnp.maximum(deg_ref[0, :] + deg_ref[1, :] + 1.0, 1.0)
    dis = lax.rsqrt(d)[:, None]
    dis_ref[...] = dis
    xs_ref[...] = x_ref[...] * dis


_BMP = 2048
_tc_pre = pl.pallas_call(
    _tc_pre_body,
    out_shape=(jax.ShapeDtypeStruct((NPAD, 1), jnp.float32),
               jax.ShapeDtypeStruct((NPAD, IN_CH), jnp.float32)),
    grid=(NPAD // _BMP,),
    in_specs=[
        pl.BlockSpec((NC, _BMP), lambda i: (0, i)),
        pl.BlockSpec((_BMP, IN_CH), lambda i: (i, 0)),
    ],
    out_specs=(pl.BlockSpec((_BMP, 1), lambda i: (i, 0)),
               pl.BlockSpec((_BMP, IN_CH), lambda i: (i, 0))),
)


def _tc_body(agg_ref, xs_ref, dis_ref, w_ref, b_ref, o_ref):
    a = (agg_ref[...] + xs_ref[...]) * dis_ref[...]
    o = jnp.dot(a, w_ref[...], preferred_element_type=jnp.float32,
                precision=lax.Precision.HIGHEST)
    o_ref[...] = jnp.maximum(o + b_ref[...], 0.0)


_BM = 1024
_tc = pl.pallas_call(
    _tc_body,
    out_shape=jax.ShapeDtypeStruct((NPAD, HID_CH), jnp.float32),
    grid=(NPAD // _BM,),
    in_specs=[
        pl.BlockSpec((_BM, IN_CH), lambda i: (i, 0)),
        pl.BlockSpec((_BM, IN_CH), lambda i: (i, 0)),
        pl.BlockSpec((_BM, 1), lambda i: (i, 0)),
        pl.BlockSpec((IN_CH, HID_CH), lambda i: (0, 0)),
        pl.BlockSpec((1, HID_CH), lambda i: (0, 0)),
    ],
    out_specs=pl.BlockSpec((_BM, HID_CH), lambda i: (i, 0)),
)


def kernel(x, edge_index, W, b):
    ei = edge_index.astype(jnp.int32)
    src, dst = ei[0], ei[1]
    x_pad = jnp.zeros((NPAD, IN_CH), x.dtype).at[:N].set(x)
    src_p = jnp.full((EPAD,), PADNODE, jnp.int32).at[:E].set(src)
    dst_p = jnp.full((EPAD,), PADNODE, jnp.int32).at[:E].set(dst)
    deg2 = _sc1(dst_p.reshape(DROWS, 128))
    dis, xs = _tc_pre(deg2, x_pad)
    agg = _sc2(src_p, dst_p, xs)
    out = _tc(agg, xs, dis, W, b.reshape(1, HID_CH))
    return out[:N]
